# baseline (speedup 1.0000x reference)
import jax
import jax.numpy as jnp
from jax import lax
from jax.experimental import pallas as pl
from jax.experimental.pallas import tpu as pltpu
from jax.experimental.pallas import tpu_sc as plsc

_NUM_ROWS = 16384
_NUM_COLS = 1000
_NUM_CORES = 2
_NUM_SUBCORES = 16
_NW = _NUM_CORES * _NUM_SUBCORES
_BAND = _NUM_ROWS // _NW
_CHUNK = 128
_NCH = _BAND // _CHUNK
_GROUPS = _CHUNK // 16


def _sc_onehot_t(idx_hbm, zeros_hbm, out_hbm, buf, idx_v, sem, isem, zsem):
    wid = lax.axis_index("s") * _NUM_CORES + lax.axis_index("c")
    band0 = wid * _BAND

    icopy = pltpu.async_copy(idx_hbm.at[pl.ds(band0, _BAND)], idx_v, isem)
    zcopy = pltpu.async_copy(zeros_hbm, buf, zsem)
    icopy.wait()
    zcopy.wait()

    iota16 = lax.iota(jnp.int32, 16)
    ones16 = jnp.full((16,), 1.0, jnp.float32)
    zerosf = jnp.zeros((16,), jnp.float32)

    def indices(c, g):
        col = idx_v[pl.ds(c * _CHUNK + g * 16, 16)]
        lane = iota16 + g * 16
        return col, lane

    copy = None
    for c in range(_NCH):
        if c >= 1:
            copy.wait()
            for g in range(_GROUPS):
                col, lane = indices(c - 1, g)
                plsc.store_scatter(buf, [col, lane], zerosf)
        for g in range(_GROUPS):
            col, lane = indices(c, g)
            plsc.store_scatter(buf, [col, lane], ones16)
        dst = out_hbm.at[:, pl.ds(band0 + c * _CHUNK, _CHUNK)]
        copy = pltpu.async_copy(buf, dst, sem)
    copy.wait()


def kernel(x, device, num_classes):
    n = x.shape[0]
    rk = jax.random.key(42)
    pred_ints = jax.random.randint(rk, (n,), 0, num_classes).astype(jnp.int32)
    zeros_chunk = jnp.zeros((_NUM_COLS, _CHUNK), jnp.float32)

    mesh = plsc.VectorSubcoreMesh(core_axis_name="c", subcore_axis_name="s")
    run = pl.kernel(
        _sc_onehot_t,
        out_type=jax.ShapeDtypeStruct((_NUM_COLS, n), jnp.float32),
        mesh=mesh,
        compiler_params=pltpu.CompilerParams(
            needs_layout_passes=False,
            use_tc_tiling_on_sc=True,
        ),
        scratch_types=[
            pltpu.VMEM((_NUM_COLS, _CHUNK), jnp.float32),
            pltpu.VMEM((_BAND,), jnp.int32),
            pltpu.SemaphoreType.DMA,
            pltpu.SemaphoreType.DMA,
            pltpu.SemaphoreType.DMA,
        ],
    )
    out_t = run(pred_ints, zeros_chunk)
    return out_t.T


# speedup vs baseline: 1.0021x; 1.0021x over previous
import jax
import jax.numpy as jnp
from jax import lax
from jax.experimental import pallas as pl
from jax.experimental.pallas import tpu as pltpu
from jax.experimental.pallas import tpu_sc as plsc

_NUM_ROWS = 16384
_NUM_COLS = 1000
_NUM_CORES = 2
_NUM_SUBCORES = 16
_NW = _NUM_CORES * _NUM_SUBCORES
_BAND = _NUM_ROWS // _NW
_CHUNK = 128
_NCH = _BAND // _CHUNK
_GROUPS = _CHUNK // 16


def _sc_onehot_t(idx_hbm, zeros_hbm, out_hbm, buf, idx_v, sem, isem, zsem):
    wid = lax.axis_index("s") * _NUM_CORES + lax.axis_index("c")
    band0 = wid * _BAND

    icopy = pltpu.async_copy(idx_hbm.at[pl.ds(band0, _BAND)], idx_v, isem)
    zcopy = pltpu.async_copy(zeros_hbm, buf, zsem)
    icopy.wait()
    zcopy.wait()

    iota16 = lax.iota(jnp.int32, 16)
    ones16 = jnp.full((16,), 1.0, jnp.float32)
    zerosf = jnp.zeros((16,), jnp.float32)

    def scatter_chunk(c, value):
        for g in range(_GROUPS):
            col = idx_v[pl.ds(c * _CHUNK + g * 16, 16)]
            lane = iota16 + g * 16
            plsc.store_scatter(buf, [col, lane], value)

    def start_dma(c):
        dst = out_hbm.at[:, pl.ds(band0 + c * _CHUNK, _CHUNK)]
        return pltpu.async_copy(buf, dst, sem)

    scatter_chunk(0, ones16)
    start_dma(0)

    def chunk_body(c, carry):
        pltpu.make_async_copy(buf, out_hbm.at[:, pl.ds(0, _CHUNK)], sem).wait()
        scatter_chunk(c - 1, zerosf)
        scatter_chunk(c, ones16)
        start_dma(c)
        return carry

    lax.fori_loop(1, _NCH, chunk_body, 0)
    pltpu.make_async_copy(buf, out_hbm.at[:, pl.ds(0, _CHUNK)], sem).wait()


def kernel(x, device, num_classes):
    n = x.shape[0]
    rk = jax.random.key(42)
    pred_ints = jax.random.randint(rk, (n,), 0, num_classes).astype(jnp.int32)
    zeros_chunk = jnp.zeros((_NUM_COLS, _CHUNK), jnp.float32)

    mesh = plsc.VectorSubcoreMesh(core_axis_name="c", subcore_axis_name="s")
    run = pl.kernel(
        _sc_onehot_t,
        out_type=jax.ShapeDtypeStruct((_NUM_COLS, n), jnp.float32),
        mesh=mesh,
        compiler_params=pltpu.CompilerParams(
            needs_layout_passes=False,
            use_tc_tiling_on_sc=True,
        ),
        scratch_types=[
            pltpu.VMEM((_NUM_COLS, _CHUNK), jnp.float32),
            pltpu.VMEM((_BAND,), jnp.int32),
            pltpu.SemaphoreType.DMA,
            pltpu.SemaphoreType.DMA,
            pltpu.SemaphoreType.DMA,
        ],
    )
    out_t = run(pred_ints, zeros_chunk)
    return out_t.T
